# Initial kernel scaffold; baseline (speedup 1.0000x reference)
import jax
import jax.numpy as jnp
from jax.experimental import pallas as pl


def kernel(x_t, h_prev, edge_index, Wz, bz, Wr, br, Wh, bh, Wz_lin, bz_lin, Wr_lin, br_lin, Wh_lin, bh_lin):
    raise NotImplementedError("write your pallas kernel here")


# speedup vs baseline: 24.2105x; 24.2105x over previous
import functools

import jax
import jax.numpy as jnp
from jax import lax
from jax.experimental import pallas as pl
from jax.experimental.pallas import tpu as pltpu
from jax.experimental.pallas import tpu_sc as plsc

NN = 10000
EE = 320000
NPAD = 10240
RPT = 640
EB = 125
ERDEG = 80
ERAGG = 80


def _deg_call(col125):
    mesh = plsc.VectorSubcoreMesh(core_axis_name="c", subcore_axis_name="s")

    @functools.partial(
        pl.kernel,
        out_type=jax.ShapeDtypeStruct((2 * NPAD,), jnp.float32),
        mesh=mesh,
        scratch_types=[
            pltpu.VMEM((ERDEG, EB), jnp.int32),
            pltpu.VMEM((RPT,), jnp.float32),
            pltpu.VMEM((128,), jnp.float32),
            pltpu.SemaphoreType.DMA,
            pltpu.VMEM_SHARED((NPAD,), jnp.float32),
        ],
    )
    def deg_kernel(col_hbm, degp_hbm, idx_v, zbuf_v, ones_v, sem, acc):
        c = lax.axis_index("c")
        s = lax.axis_index("s")
        w = s * 2 + c

        def zrow(i, _):
            zbuf_v[pl.ds(i * 16, 16)] = jnp.zeros((16,), jnp.float32)
            return 0

        lax.fori_loop(0, RPT // 16, zrow, 0)

        def orow(i, _):
            ones_v[pl.ds(i * 16, 16)] = jnp.ones((16,), jnp.float32)
            return 0

        lax.fori_loop(0, 8, orow, 0)

        pltpu.sync_copy(zbuf_v, acc.at[pl.ds(s * RPT, RPT)])
        plsc.subcore_barrier()

        pltpu.sync_copy(col_hbm.at[pl.ds(w * ERDEG, ERDEG)], idx_v)
        copies = [
            pltpu.async_copy(ones_v.at[pl.ds(0, EB)], acc.at[idx_v.at[b]],
                             sem, add=True)
            for b in range(ERDEG)
        ]
        for cp in copies:
            cp.wait()
        plsc.subcore_barrier()
        pltpu.sync_copy(acc.at[pl.ds(s * RPT, RPT)],
                        degp_hbm.at[pl.ds(c * NPAD + s * RPT, RPT)])

    return deg_kernel(col125)


def _scale_matmul_call(xp, wcat, degp):
    def body(x_ref, w_ref, da_ref, db_ref, o0_ref, o1_ref, o2_ref):
        deg = da_ref[...] + db_ref[...] + 1.0
        dis = lax.rsqrt(deg)
        res = jnp.dot(x_ref[...], w_ref[...],
                      preferred_element_type=jnp.float32)
        res = res * dis
        o0_ref[...] = res[:, 0:128]
        o1_ref[...] = res[:, 128:256]
        o2_ref[...] = res[:, 256:384]

    grid = NPAD // RPT
    rb = lambda i: (i, 0)
    return pl.pallas_call(
        body,
        grid=(grid,),
        in_specs=[
            pl.BlockSpec((RPT, 128), rb),
            pl.BlockSpec((128, 384), lambda i: (0, 0)),
            pl.BlockSpec((RPT, 1), rb),
            pl.BlockSpec((RPT, 1), lambda i: (i + grid, 0)),
        ],
        out_specs=[
            pl.BlockSpec((RPT, 128), rb),
            pl.BlockSpec((RPT, 128), rb),
            pl.BlockSpec((RPT, 128), rb),
        ],
        out_shape=[
            jax.ShapeDtypeStruct((NPAD, 128), jnp.float32),
            jax.ShapeDtypeStruct((NPAD, 128), jnp.float32),
            jax.ShapeDtypeStruct((NPAD, 128), jnp.float32),
        ],
    )(xp, wcat, degp, degp)


def _agg_call(xs0, xs1, xs2, row125, col125):
    mesh = plsc.VectorSubcoreMesh(core_axis_name="c", subcore_axis_name="s")

    @functools.partial(
        pl.kernel,
        out_type=jax.ShapeDtypeStruct((6 * NPAD, 128), jnp.float32),
        mesh=mesh,
        scratch_types=[
            pltpu.VMEM((ERAGG // 2, EB), jnp.int32),
            pltpu.VMEM((ERAGG // 2, EB), jnp.int32),
            pltpu.VMEM((EB, 128), jnp.float32),
            pltpu.VMEM((EB, 128), jnp.float32),
            pltpu.SemaphoreType.DMA,
            pltpu.SemaphoreType.DMA,
            pltpu.VMEM_SHARED((NPAD, 128), jnp.float32),
        ],
    )
    def agg_kernel(x0_hbm, x1_hbm, x2_hbm, row_hbm, col_hbm, t_hbm,
                   ridx, cidx, gb0, gb1, sem0, sem1, acc):
        c = lax.axis_index("c")
        s = lax.axis_index("s")
        ebase = c * (16 * ERAGG) + s * ERAGG
        nb = ERAGG // 2

        for j, xs in enumerate((x0_hbm, x1_hbm, x2_hbm)):
            def zrow(i, _):
                def zcol(k, _):
                    gb0[i, pl.ds(k * 16, 16)] = jnp.zeros((16,),
                                                          jnp.float32)
                    return 0
                lax.fori_loop(0, 8, zcol, 0)
                return 0

            lax.fori_loop(0, EB, zrow, 0)
            for k in range(RPT // 80):
                pltpu.sync_copy(gb0.at[pl.ds(0, 80)],
                                acc.at[pl.ds(s * RPT + k * 80, 80)])
            plsc.subcore_barrier()

            for h in range(2):
                pltpu.sync_copy(row_hbm.at[pl.ds(ebase + h * nb, nb)], ridx)
                pltpu.sync_copy(col_hbm.at[pl.ds(ebase + h * nb, nb)], cidx)
                pltpu.async_copy(xs.at[ridx.at[0]], gb0, sem0)

                def step(g, _):
                    b0 = g * 2
                    pltpu.make_async_copy(xs.at[ridx.at[b0]], gb0,
                                          sem0).wait()

                    @pl.when(b0 + 1 < nb)
                    def _():
                        pltpu.async_copy(xs.at[ridx.at[b0 + 1]], gb1, sem1)

                    pltpu.sync_copy(gb0, acc.at[cidx.at[b0]], add=True)

                    b1 = b0 + 1
                    pltpu.make_async_copy(xs.at[ridx.at[b1]], gb1,
                                          sem1).wait()

                    @pl.when(b1 + 1 < nb)
                    def _():
                        pltpu.async_copy(xs.at[ridx.at[b1 + 1]], gb0, sem0)

                    pltpu.sync_copy(gb1, acc.at[cidx.at[b1]], add=True)
                    return 0

                lax.fori_loop(0, nb // 2, step, 0)

            plsc.subcore_barrier()
            obase = (2 * j + c) * NPAD + s * RPT
            pltpu.sync_copy(acc.at[pl.ds(s * RPT, RPT)],
                            t_hbm.at[pl.ds(obase, RPT)])

    return agg_kernel(xs0, xs1, xs2, row125, col125)


def _gate_call(tt, xs0, xs1, xs2, degp, hp,
               bz2, br2, bh2, wzl, wrl, whl, bzl2, brl2, bhl2):
    def body(t0a, t0b, t1a, t1b, t2a, t2b, x0, x1, x2, da, db, h,
             bz_r, br_r, bh_r, wz_r, wr_r, wh_r, bzl_r, brl_r, bhl_r, out):
        deg = da[...] + db[...] + 1.0
        dis = lax.rsqrt(deg)
        h_ = h[...]
        cz = dis * (t0a[...] + t0b[...] + x0[...]) + bz_r[...]
        cr = dis * (t1a[...] + t1b[...] + x1[...]) + br_r[...]
        ch = dis * (t2a[...] + t2b[...] + x2[...]) + bh_r[...]
        wz = wz_r[...]
        wr = wr_r[...]
        wh = wh_r[...]
        dot = lambda a, b: jnp.dot(a, b, preferred_element_type=jnp.float32)
        z = jax.nn.sigmoid(dot(cz, wz[:128]) + dot(h_, wz[128:]) + bzl_r[...])
        r = jax.nn.sigmoid(dot(cr, wr[:128]) + dot(h_, wr[128:]) + brl_r[...])
        ht = jnp.tanh(dot(ch, wh[:128]) + dot(h_ * r, wh[128:]) + bhl_r[...])
        out[...] = z * h_ + (1.0 - z) * ht

    grid = NPAD // RPT
    rb = lambda i: (i, 0)
    full = lambda i: (0, 0)
    tspec = lambda k: pl.BlockSpec((RPT, 128), lambda i, k=k: (i + k * grid, 0))
    return pl.pallas_call(
        body,
        grid=(grid,),
        in_specs=[
            tspec(0), tspec(1), tspec(2), tspec(3), tspec(4), tspec(5),
            pl.BlockSpec((RPT, 128), rb),
            pl.BlockSpec((RPT, 128), rb),
            pl.BlockSpec((RPT, 128), rb),
            pl.BlockSpec((RPT, 1), rb),
            pl.BlockSpec((RPT, 1), lambda i: (i + grid, 0)),
            pl.BlockSpec((RPT, 128), rb),
            pl.BlockSpec((1, 128), full),
            pl.BlockSpec((1, 128), full),
            pl.BlockSpec((1, 128), full),
            pl.BlockSpec((256, 128), full),
            pl.BlockSpec((256, 128), full),
            pl.BlockSpec((256, 128), full),
            pl.BlockSpec((1, 128), full),
            pl.BlockSpec((1, 128), full),
            pl.BlockSpec((1, 128), full),
        ],
        out_specs=pl.BlockSpec((RPT, 128), rb),
        out_shape=jax.ShapeDtypeStruct((NN, 128), jnp.float32),
    )(tt, tt, tt, tt, tt, tt, xs0, xs1, xs2, degp, degp, hp,
      bz2, br2, bh2, wzl, wrl, whl, bzl2, brl2, bhl2)


def kernel(x_t, h_prev, edge_index, Wz, bz, Wr, br, Wh, bh,
           Wz_lin, bz_lin, Wr_lin, br_lin, Wh_lin, bh_lin):
    row = edge_index[0]
    col = edge_index[1]
    row125 = row.reshape(EE // EB, EB)
    col125 = col.reshape(EE // EB, EB)
    wcat = jnp.concatenate([Wz, Wr, Wh], axis=1)
    xp = jnp.pad(x_t, ((0, NPAD - NN), (0, 0)))
    hp = jnp.pad(h_prev, ((0, NPAD - NN), (0, 0)))

    degp = _deg_call(col125).reshape(2 * NPAD, 1)
    xs0, xs1, xs2 = _scale_matmul_call(xp, wcat, degp)
    tt = _agg_call(xs0, xs1, xs2, row125, col125)
    return _gate_call(
        tt, xs0, xs1, xs2, degp, hp,
        bz.reshape(1, 128), br.reshape(1, 128), bh.reshape(1, 128),
        Wz_lin, Wr_lin, Wh_lin,
        bz_lin.reshape(1, 128), br_lin.reshape(1, 128),
        bh_lin.reshape(1, 128))


# speedup vs baseline: 26.4203x; 1.0913x over previous
import functools

import jax
import jax.numpy as jnp
from jax import lax
from jax.experimental import pallas as pl
from jax.experimental.pallas import tpu as pltpu
from jax.experimental.pallas import tpu_sc as plsc

NN = 10000
EE = 320000
NPAD = 10240
RPT = 640
EB = 125
ERDEG = 80
ERAGG = 80
EBA = 50


def _deg_call(col125):
    mesh = plsc.VectorSubcoreMesh(core_axis_name="c", subcore_axis_name="s")

    @functools.partial(
        pl.kernel,
        out_type=jax.ShapeDtypeStruct((2 * NPAD,), jnp.float32),
        mesh=mesh,
        scratch_types=[
            pltpu.VMEM((ERDEG, EB), jnp.int32),
            pltpu.VMEM((RPT,), jnp.float32),
            pltpu.VMEM((128,), jnp.float32),
            pltpu.SemaphoreType.DMA,
            pltpu.VMEM_SHARED((NPAD,), jnp.float32),
        ],
    )
    def deg_kernel(col_hbm, degp_hbm, idx_v, zbuf_v, ones_v, sem, acc):
        c = lax.axis_index("c")
        s = lax.axis_index("s")
        w = s * 2 + c

        def zrow(i, _):
            zbuf_v[pl.ds(i * 16, 16)] = jnp.zeros((16,), jnp.float32)
            return 0

        lax.fori_loop(0, RPT // 16, zrow, 0)

        def orow(i, _):
            ones_v[pl.ds(i * 16, 16)] = jnp.ones((16,), jnp.float32)
            return 0

        lax.fori_loop(0, 8, orow, 0)

        pltpu.sync_copy(zbuf_v, acc.at[pl.ds(s * RPT, RPT)])
        plsc.subcore_barrier()

        pltpu.sync_copy(col_hbm.at[pl.ds(w * ERDEG, ERDEG)], idx_v)
        copies = [
            pltpu.async_copy(ones_v.at[pl.ds(0, EB)], acc.at[idx_v.at[b]],
                             sem, add=True)
            for b in range(ERDEG)
        ]
        for cp in copies:
            cp.wait()
        plsc.subcore_barrier()
        pltpu.sync_copy(acc.at[pl.ds(s * RPT, RPT)],
                        degp_hbm.at[pl.ds(c * NPAD + s * RPT, RPT)])

    return deg_kernel(col125)


def _scale_matmul_call(xp, wcat, degp):
    def body(x_ref, w_ref, da_ref, db_ref, o0_ref, o1_ref, o2_ref):
        deg = da_ref[...] + db_ref[...] + 1.0
        dis = lax.rsqrt(deg)
        res = jnp.dot(x_ref[...], w_ref[...],
                      preferred_element_type=jnp.float32)
        res = res * dis
        o0_ref[...] = res[:, 0:128]
        o1_ref[...] = res[:, 128:256]
        o2_ref[...] = res[:, 256:384]

    grid = NPAD // RPT
    rb = lambda i: (i, 0)
    return pl.pallas_call(
        body,
        grid=(grid,),
        in_specs=[
            pl.BlockSpec((RPT, 128), rb),
            pl.BlockSpec((128, 384), lambda i: (0, 0)),
            pl.BlockSpec((RPT, 1), rb),
            pl.BlockSpec((RPT, 1), lambda i: (i + grid, 0)),
        ],
        out_specs=[
            pl.BlockSpec((RPT, 128), rb),
            pl.BlockSpec((RPT, 128), rb),
            pl.BlockSpec((RPT, 128), rb),
        ],
        out_shape=[
            jax.ShapeDtypeStruct((NPAD, 128), jnp.float32),
            jax.ShapeDtypeStruct((NPAD, 128), jnp.float32),
            jax.ShapeDtypeStruct((NPAD, 128), jnp.float32),
        ],
    )(xp, wcat, degp, degp)


def _agg_call(xs0, xs1, xs2, row50, col50):
    mesh = plsc.VectorSubcoreMesh(core_axis_name="c", subcore_axis_name="s")
    NB = EE // EBA // 2 // 16
    CH = 40

    @functools.partial(
        pl.kernel,
        out_type=jax.ShapeDtypeStruct((6 * NPAD, 128), jnp.float32),
        mesh=mesh,
        scratch_types=[
            pltpu.VMEM((CH, EBA), jnp.int32),
            pltpu.VMEM((CH, EBA), jnp.int32),
            [pltpu.VMEM((EBA, 128), jnp.float32) for _ in range(4)],
            [pltpu.SemaphoreType.DMA for _ in range(4)],
            [pltpu.SemaphoreType.DMA for _ in range(4)],
            pltpu.VMEM_SHARED((NPAD, 128), jnp.float32),
        ],
    )
    def agg_kernel(x0_hbm, x1_hbm, x2_hbm, row_hbm, col_hbm, t_hbm,
                   ridx, cidx, gb, gsem, ssem, acc):
        c = lax.axis_index("c")
        s = lax.axis_index("s")
        ebase = c * (16 * NB) + s * NB

        for j, xs in enumerate((x0_hbm, x1_hbm, x2_hbm)):
            def zrow(i, _):
                def zcol(k, _):
                    gb[0][i, pl.ds(k * 16, 16)] = jnp.zeros((16,),
                                                            jnp.float32)
                    return 0
                lax.fori_loop(0, 8, zcol, 0)
                return 0

            lax.fori_loop(0, EBA, zrow, 0)
            for k in range(RPT // 40):
                pltpu.sync_copy(gb[0].at[pl.ds(0, 40)],
                                acc.at[pl.ds(s * RPT + k * 40, 40)])
            plsc.subcore_barrier()

            for h in range(NB // CH):
                pltpu.sync_copy(row_hbm.at[pl.ds(ebase + h * CH, CH)],
                                ridx)
                pltpu.sync_copy(col_hbm.at[pl.ds(ebase + h * CH, CH)],
                                cidx)
                for b in range(3):
                    pltpu.async_copy(xs.at[ridx.at[b]], gb[b], gsem[b])

                def step(g, _):
                    for u in range(4):
                        b = g * 4 + u
                        pltpu.make_async_copy(xs.at[ridx.at[b]], gb[u],
                                              gsem[u]).wait()
                        pltpu.async_copy(gb[u], acc.at[cidx.at[b]],
                                         ssem[u], add=True)
                        un = (u + 3) % 4

                        @pl.when(b + 3 < CH)
                        def _():
                            @pl.when(b >= 1)
                            def _():
                                pltpu.make_async_copy(
                                    gb[un], acc.at[cidx.at[b]],
                                    ssem[un]).wait()

                            pltpu.async_copy(xs.at[ridx.at[b + 3]],
                                             gb[un], gsem[un])
                    return 0

                lax.fori_loop(0, CH // 4, step, 0)
                for u in range(4):
                    pltpu.make_async_copy(gb[u], acc.at[cidx.at[0]],
                                          ssem[u]).wait()

            plsc.subcore_barrier()
            obase = (2 * j + c) * NPAD + s * RPT
            pltpu.sync_copy(acc.at[pl.ds(s * RPT, RPT)],
                            t_hbm.at[pl.ds(obase, RPT)])

    return agg_kernel(xs0, xs1, xs2, row50, col50)


def _gate_call(tt, xs0, xs1, xs2, degp, hp,
               bz2, br2, bh2, wzl, wrl, whl, bzl2, brl2, bhl2):
    def body(t0a, t0b, t1a, t1b, t2a, t2b, x0, x1, x2, da, db, h,
             bz_r, br_r, bh_r, wz_r, wr_r, wh_r, bzl_r, brl_r, bhl_r, out):
        deg = da[...] + db[...] + 1.0
        dis = lax.rsqrt(deg)
        h_ = h[...]
        f32 = lambda v: v[...].astype(jnp.float32)
        cz = dis * (f32(t0a) + f32(t0b) + f32(x0)) + bz_r[...]
        cr = dis * (f32(t1a) + f32(t1b) + f32(x1)) + br_r[...]
        ch = dis * (f32(t2a) + f32(t2b) + f32(x2)) + bh_r[...]
        wz = wz_r[...]
        wr = wr_r[...]
        wh = wh_r[...]
        dot = lambda a, b: jnp.dot(a, b, preferred_element_type=jnp.float32)
        z = jax.nn.sigmoid(dot(cz, wz[:128]) + dot(h_, wz[128:]) + bzl_r[...])
        r = jax.nn.sigmoid(dot(cr, wr[:128]) + dot(h_, wr[128:]) + brl_r[...])
        ht = jnp.tanh(dot(ch, wh[:128]) + dot(h_ * r, wh[128:]) + bhl_r[...])
        out[...] = z * h_ + (1.0 - z) * ht

    grid = NPAD // RPT
    rb = lambda i: (i, 0)
    full = lambda i: (0, 0)
    tspec = lambda k: pl.BlockSpec((RPT, 128), lambda i, k=k: (i + k * grid, 0))
    return pl.pallas_call(
        body,
        grid=(grid,),
        in_specs=[
            tspec(0), tspec(1), tspec(2), tspec(3), tspec(4), tspec(5),
            pl.BlockSpec((RPT, 128), rb),
            pl.BlockSpec((RPT, 128), rb),
            pl.BlockSpec((RPT, 128), rb),
            pl.BlockSpec((RPT, 1), rb),
            pl.BlockSpec((RPT, 1), lambda i: (i + grid, 0)),
            pl.BlockSpec((RPT, 128), rb),
            pl.BlockSpec((1, 128), full),
            pl.BlockSpec((1, 128), full),
            pl.BlockSpec((1, 128), full),
            pl.BlockSpec((256, 128), full),
            pl.BlockSpec((256, 128), full),
            pl.BlockSpec((256, 128), full),
            pl.BlockSpec((1, 128), full),
            pl.BlockSpec((1, 128), full),
            pl.BlockSpec((1, 128), full),
        ],
        out_specs=pl.BlockSpec((RPT, 128), rb),
        out_shape=jax.ShapeDtypeStruct((NN, 128), jnp.float32),
    )(tt, tt, tt, tt, tt, tt, xs0, xs1, xs2, degp, degp, hp,
      bz2, br2, bh2, wzl, wrl, whl, bzl2, brl2, bhl2)


def kernel(x_t, h_prev, edge_index, Wz, bz, Wr, br, Wh, bh,
           Wz_lin, bz_lin, Wr_lin, br_lin, Wh_lin, bh_lin):
    row = edge_index[0]
    col = edge_index[1]
    col125 = col.reshape(EE // EB, EB)
    row50 = row.reshape(EE // EBA, EBA)
    col50 = col.reshape(EE // EBA, EBA)
    wcat = jnp.concatenate([Wz, Wr, Wh], axis=1)
    xp = jnp.pad(x_t, ((0, NPAD - NN), (0, 0)))
    hp = jnp.pad(h_prev, ((0, NPAD - NN), (0, 0)))

    degp = _deg_call(col125).reshape(2 * NPAD, 1)
    xs0, xs1, xs2 = _scale_matmul_call(xp, wcat, degp)
    tt = _agg_call(xs0, xs1, xs2, row50, col50)
    return _gate_call(
        tt, xs0, xs1, xs2, degp, hp,
        bz.reshape(1, 128), br.reshape(1, 128), bh.reshape(1, 128),
        Wz_lin, Wr_lin, Wh_lin,
        bz_lin.reshape(1, 128), br_lin.reshape(1, 128),
        bh_lin.reshape(1, 128))


# speedup vs baseline: 26.4475x; 1.0010x over previous
import functools

import jax
import jax.numpy as jnp
from jax import lax
from jax.experimental import pallas as pl
from jax.experimental.pallas import tpu as pltpu
from jax.experimental.pallas import tpu_sc as plsc

NN = 10000
EE = 320000
NPAD = 10240
RPT = 640
EB = 125
ERDEG = 80
ERAGG = 80
EBA = 50


def _deg_call(col125):
    mesh = plsc.VectorSubcoreMesh(core_axis_name="c", subcore_axis_name="s")

    @functools.partial(
        pl.kernel,
        out_type=jax.ShapeDtypeStruct((2 * NPAD,), jnp.float32),
        mesh=mesh,
        scratch_types=[
            pltpu.VMEM((ERDEG, EB), jnp.int32),
            pltpu.VMEM((RPT,), jnp.float32),
            pltpu.VMEM((128,), jnp.float32),
            pltpu.SemaphoreType.DMA,
            pltpu.VMEM_SHARED((NPAD,), jnp.float32),
        ],
    )
    def deg_kernel(col_hbm, degp_hbm, idx_v, zbuf_v, ones_v, sem, acc):
        c = lax.axis_index("c")
        s = lax.axis_index("s")
        w = s * 2 + c

        def zrow(i, _):
            zbuf_v[pl.ds(i * 16, 16)] = jnp.zeros((16,), jnp.float32)
            return 0

        lax.fori_loop(0, RPT // 16, zrow, 0)

        def orow(i, _):
            ones_v[pl.ds(i * 16, 16)] = jnp.ones((16,), jnp.float32)
            return 0

        lax.fori_loop(0, 8, orow, 0)

        pltpu.sync_copy(zbuf_v, acc.at[pl.ds(s * RPT, RPT)])
        plsc.subcore_barrier()

        pltpu.sync_copy(col_hbm.at[pl.ds(w * ERDEG, ERDEG)], idx_v)
        copies = [
            pltpu.async_copy(ones_v.at[pl.ds(0, EB)], acc.at[idx_v.at[b]],
                             sem, add=True)
            for b in range(ERDEG)
        ]
        for cp in copies:
            cp.wait()
        plsc.subcore_barrier()
        pltpu.sync_copy(acc.at[pl.ds(s * RPT, RPT)],
                        degp_hbm.at[pl.ds(c * NPAD + s * RPT, RPT)])

    return deg_kernel(col125)


def _scale_matmul_call(xp, wcat, degp):
    def body(x_ref, w_ref, da_ref, db_ref, o0_ref, o1_ref, o2_ref):
        deg = da_ref[...] + db_ref[...] + 1.0
        dis = lax.rsqrt(deg)
        res = jnp.dot(x_ref[...], w_ref[...],
                      preferred_element_type=jnp.float32)
        res = res * dis
        o0_ref[...] = res[:, 0:128]
        o1_ref[...] = res[:, 128:256]
        o2_ref[...] = res[:, 256:384]

    grid = NPAD // RPT
    rb = lambda i: (i, 0)
    return pl.pallas_call(
        body,
        grid=(grid,),
        in_specs=[
            pl.BlockSpec((RPT, 128), rb),
            pl.BlockSpec((128, 384), lambda i: (0, 0)),
            pl.BlockSpec((RPT, 1), rb),
            pl.BlockSpec((RPT, 1), lambda i: (i + grid, 0)),
        ],
        out_specs=[
            pl.BlockSpec((RPT, 128), rb),
            pl.BlockSpec((RPT, 128), rb),
            pl.BlockSpec((RPT, 128), rb),
        ],
        out_shape=[
            jax.ShapeDtypeStruct((NPAD, 128), jnp.float32),
            jax.ShapeDtypeStruct((NPAD, 128), jnp.float32),
            jax.ShapeDtypeStruct((NPAD, 128), jnp.float32),
        ],
    )(xp, wcat, degp, degp)


def _agg_call(xs0, xs1, xs2, row50, col50):
    mesh = plsc.VectorSubcoreMesh(core_axis_name="c", subcore_axis_name="s")
    NB = EE // EBA // 2 // 16
    CH = 40
    NBUF = 5

    @functools.partial(
        pl.kernel,
        out_type=jax.ShapeDtypeStruct((6 * NPAD, 128), jnp.float32),
        mesh=mesh,
        scratch_types=[
            pltpu.VMEM((CH, EBA), jnp.int32),
            pltpu.VMEM((CH, EBA), jnp.int32),
            [pltpu.VMEM((EBA, 128), jnp.float32) for _ in range(NBUF)],
            [pltpu.SemaphoreType.DMA for _ in range(NBUF)],
            [pltpu.SemaphoreType.DMA for _ in range(NBUF)],
            pltpu.VMEM_SHARED((NPAD, 128), jnp.float32),
        ],
    )
    def agg_kernel(x0_hbm, x1_hbm, x2_hbm, row_hbm, col_hbm, t_hbm,
                   ridx, cidx, gb, gsem, ssem, acc):
        c = lax.axis_index("c")
        s = lax.axis_index("s")
        ebase = c * (16 * NB) + s * NB

        for j, xs in enumerate((x0_hbm, x1_hbm, x2_hbm)):
            def zrow(i, _):
                def zcol(k, _):
                    gb[0][i, pl.ds(k * 16, 16)] = jnp.zeros((16,),
                                                            jnp.float32)
                    return 0
                lax.fori_loop(0, 8, zcol, 0)
                return 0

            lax.fori_loop(0, EBA, zrow, 0)
            for k in range(RPT // 40):
                pltpu.sync_copy(gb[0].at[pl.ds(0, 40)],
                                acc.at[pl.ds(s * RPT + k * 40, 40)])
            plsc.subcore_barrier()

            for h in range(NB // CH):
                pltpu.sync_copy(row_hbm.at[pl.ds(ebase + h * CH, CH)],
                                ridx)
                pltpu.sync_copy(col_hbm.at[pl.ds(ebase + h * CH, CH)],
                                cidx)
                for b in range(3):
                    pltpu.async_copy(xs.at[ridx.at[b]], gb[b], gsem[b])

                def step(g, _):
                    for u in range(4):
                        b = g * 4 + u
                        pltpu.make_async_copy(xs.at[ridx.at[b]], gb[u],
                                              gsem[u]).wait()
                        pltpu.async_copy(gb[u], acc.at[cidx.at[b]],
                                         ssem[u], add=True)
                        un = (u + 3) % 4

                        @pl.when(b + 3 < CH)
                        def _():
                            @pl.when(b >= 1)
                            def _():
                                pltpu.make_async_copy(
                                    gb[un], acc.at[cidx.at[b]],
                                    ssem[un]).wait()

                            pltpu.async_copy(xs.at[ridx.at[b + 3]],
                                             gb[un], gsem[un])
                    return 0

                lax.fori_loop(0, CH // 4, step, 0)
                for u in range(4):
                    pltpu.make_async_copy(gb[u], acc.at[cidx.at[0]],
                                          ssem[u]).wait()

            plsc.subcore_barrier()
            obase = (2 * j + c) * NPAD + s * RPT
            pltpu.sync_copy(acc.at[pl.ds(s * RPT, RPT)],
                            t_hbm.at[pl.ds(obase, RPT)])

    return agg_kernel(xs0, xs1, xs2, row50, col50)


def _gate_call(tt, xs0, xs1, xs2, degp, hp,
               bz2, br2, bh2, wzl, wrl, whl, bzl2, brl2, bhl2):
    def body(t0a, t0b, t1a, t1b, t2a, t2b, x0, x1, x2, da, db, h,
             bz_r, br_r, bh_r, wz_r, wr_r, wh_r, bzl_r, brl_r, bhl_r, out):
        deg = da[...] + db[...] + 1.0
        dis = lax.rsqrt(deg)
        h_ = h[...]
        f32 = lambda v: v[...].astype(jnp.float32)
        cz = dis * (f32(t0a) + f32(t0b) + f32(x0)) + bz_r[...]
        cr = dis * (f32(t1a) + f32(t1b) + f32(x1)) + br_r[...]
        ch = dis * (f32(t2a) + f32(t2b) + f32(x2)) + bh_r[...]
        wz = wz_r[...]
        wr = wr_r[...]
        wh = wh_r[...]
        dot = lambda a, b: jnp.dot(a, b, preferred_element_type=jnp.float32)
        z = jax.nn.sigmoid(dot(cz, wz[:128]) + dot(h_, wz[128:]) + bzl_r[...])
        r = jax.nn.sigmoid(dot(cr, wr[:128]) + dot(h_, wr[128:]) + brl_r[...])
        ht = jnp.tanh(dot(ch, wh[:128]) + dot(h_ * r, wh[128:]) + bhl_r[...])
        out[...] = z * h_ + (1.0 - z) * ht

    grid = NPAD // RPT
    rb = lambda i: (i, 0)
    full = lambda i: (0, 0)
    tspec = lambda k: pl.BlockSpec((RPT, 128), lambda i, k=k: (i + k * grid, 0))
    return pl.pallas_call(
        body,
        grid=(grid,),
        in_specs=[
            tspec(0), tspec(1), tspec(2), tspec(3), tspec(4), tspec(5),
            pl.BlockSpec((RPT, 128), rb),
            pl.BlockSpec((RPT, 128), rb),
            pl.BlockSpec((RPT, 128), rb),
            pl.BlockSpec((RPT, 1), rb),
            pl.BlockSpec((RPT, 1), lambda i: (i + grid, 0)),
            pl.BlockSpec((RPT, 128), rb),
            pl.BlockSpec((1, 128), full),
            pl.BlockSpec((1, 128), full),
            pl.BlockSpec((1, 128), full),
            pl.BlockSpec((256, 128), full),
            pl.BlockSpec((256, 128), full),
            pl.BlockSpec((256, 128), full),
            pl.BlockSpec((1, 128), full),
            pl.BlockSpec((1, 128), full),
            pl.BlockSpec((1, 128), full),
        ],
        out_specs=pl.BlockSpec((RPT, 128), rb),
        out_shape=jax.ShapeDtypeStruct((NN, 128), jnp.float32),
    )(tt, tt, tt, tt, tt, tt, xs0, xs1, xs2, degp, degp, hp,
      bz2, br2, bh2, wzl, wrl, whl, bzl2, brl2, bhl2)


def kernel(x_t, h_prev, edge_index, Wz, bz, Wr, br, Wh, bh,
           Wz_lin, bz_lin, Wr_lin, br_lin, Wh_lin, bh_lin):
    row = edge_index[0]
    col = edge_index[1]
    col125 = col.reshape(EE // EB, EB)
    row50 = row.reshape(EE // EBA, EBA)
    col50 = col.reshape(EE // EBA, EBA)
    wcat = jnp.concatenate([Wz, Wr, Wh], axis=1)
    xp = jnp.pad(x_t, ((0, NPAD - NN), (0, 0)))
    hp = jnp.pad(h_prev, ((0, NPAD - NN), (0, 0)))

    degp = _deg_call(col125).reshape(2 * NPAD, 1)
    xs0, xs1, xs2 = _scale_matmul_call(xp, wcat, degp)
    tt = _agg_call(xs0, xs1, xs2, row50, col50)
    return _gate_call(
        tt, xs0, xs1, xs2, degp, hp,
        bz.reshape(1, 128), br.reshape(1, 128), bh.reshape(1, 128),
        Wz_lin, Wr_lin, Wh_lin,
        bz_lin.reshape(1, 128), br_lin.reshape(1, 128),
        bh_lin.reshape(1, 128))


# speedup vs baseline: 26.7348x; 1.0109x over previous
import functools

import jax
import jax.numpy as jnp
from jax import lax
from jax.experimental import pallas as pl
from jax.experimental.pallas import tpu as pltpu
from jax.experimental.pallas import tpu_sc as plsc

NN = 10000
EE = 320000
NPAD = 10240
RPT = 640
EB = 125
ERDEG = 80
ERAGG = 80
EBA = 50


def _deg_call(col125):
    mesh = plsc.VectorSubcoreMesh(core_axis_name="c", subcore_axis_name="s")

    @functools.partial(
        pl.kernel,
        out_type=jax.ShapeDtypeStruct((2 * NPAD,), jnp.float32),
        mesh=mesh,
        scratch_types=[
            pltpu.VMEM((ERDEG, EB), jnp.int32),
            pltpu.VMEM((RPT,), jnp.float32),
            pltpu.VMEM((128,), jnp.float32),
            pltpu.SemaphoreType.DMA,
            pltpu.VMEM_SHARED((NPAD,), jnp.float32),
        ],
    )
    def deg_kernel(col_hbm, degp_hbm, idx_v, zbuf_v, ones_v, sem, acc):
        c = lax.axis_index("c")
        s = lax.axis_index("s")
        w = s * 2 + c

        def zrow(i, _):
            zbuf_v[pl.ds(i * 16, 16)] = jnp.zeros((16,), jnp.float32)
            return 0

        lax.fori_loop(0, RPT // 16, zrow, 0)

        def orow(i, _):
            ones_v[pl.ds(i * 16, 16)] = jnp.ones((16,), jnp.float32)
            return 0

        lax.fori_loop(0, 8, orow, 0)

        pltpu.sync_copy(zbuf_v, acc.at[pl.ds(s * RPT, RPT)])
        plsc.subcore_barrier()

        pltpu.sync_copy(col_hbm.at[pl.ds(w * ERDEG, ERDEG)], idx_v)
        copies = [
            pltpu.async_copy(ones_v.at[pl.ds(0, EB)], acc.at[idx_v.at[b]],
                             sem, add=True)
            for b in range(ERDEG)
        ]
        for cp in copies:
            cp.wait()
        plsc.subcore_barrier()
        pltpu.sync_copy(acc.at[pl.ds(s * RPT, RPT)],
                        degp_hbm.at[pl.ds(c * NPAD + s * RPT, RPT)])

    return deg_kernel(col125)


def _scale_matmul_call(xp, wcat, degp):
    def body(x_ref, w_ref, da_ref, db_ref, o0_ref, o1_ref, o2_ref):
        deg = da_ref[...] + db_ref[...] + 1.0
        dis = lax.rsqrt(deg)
        res = jnp.dot(x_ref[...], w_ref[...],
                      preferred_element_type=jnp.float32)
        res = res * dis
        o0_ref[...] = res[:, 0:128]
        o1_ref[...] = res[:, 128:256]
        o2_ref[...] = res[:, 256:384]

    grid = NPAD // RPT
    rb = lambda i: (i, 0)
    return pl.pallas_call(
        body,
        grid=(grid,),
        in_specs=[
            pl.BlockSpec((RPT, 128), rb),
            pl.BlockSpec((128, 384), lambda i: (0, 0)),
            pl.BlockSpec((RPT, 1), rb),
            pl.BlockSpec((RPT, 1), lambda i: (i + grid, 0)),
        ],
        out_specs=[
            pl.BlockSpec((RPT, 128), rb),
            pl.BlockSpec((RPT, 128), rb),
            pl.BlockSpec((RPT, 128), rb),
        ],
        out_shape=[
            jax.ShapeDtypeStruct((NPAD, 128), jnp.float32),
            jax.ShapeDtypeStruct((NPAD, 128), jnp.float32),
            jax.ShapeDtypeStruct((NPAD, 128), jnp.float32),
        ],
    )(xp, wcat, degp, degp)


def _agg_call(xs0, xs1, xs2, row50, col50):
    mesh = plsc.VectorSubcoreMesh(core_axis_name="c", subcore_axis_name="s")
    NB = EE // EBA // 2 // 16
    CH = 40
    NBUF = 5

    @functools.partial(
        pl.kernel,
        out_type=jax.ShapeDtypeStruct((6 * NPAD, 128), jnp.float32),
        mesh=mesh,
        scratch_types=[
            pltpu.VMEM((CH, EBA), jnp.int32),
            pltpu.VMEM((CH, EBA), jnp.int32),
            [pltpu.VMEM((EBA, 128), jnp.float32) for _ in range(NBUF)],
            [pltpu.SemaphoreType.DMA for _ in range(NBUF)],
            [pltpu.SemaphoreType.DMA for _ in range(NBUF)],
            pltpu.VMEM_SHARED((NPAD, 128), jnp.float32),
        ],
    )
    def agg_kernel(x0_hbm, x1_hbm, x2_hbm, row_hbm, col_hbm, t_hbm,
                   ridx, cidx, gb, gsem, ssem, acc):
        c = lax.axis_index("c")
        s = lax.axis_index("s")
        ebase = c * (16 * NB) + s * NB

        for j, xs in enumerate((x0_hbm, x1_hbm, x2_hbm)):
            def zrow(i, _):
                def zcol(k, _):
                    gb[0][i, pl.ds(k * 16, 16)] = jnp.zeros((16,),
                                                            jnp.float32)
                    return 0
                lax.fori_loop(0, 8, zcol, 0)
                return 0

            lax.fori_loop(0, EBA, zrow, 0)
            for k in range(RPT // 40):
                pltpu.sync_copy(gb[0].at[pl.ds(0, 40)],
                                acc.at[pl.ds(s * RPT + k * 40, 40)])
            plsc.subcore_barrier()

            for h in range(NB // CH):
                pltpu.sync_copy(row_hbm.at[pl.ds(ebase + h * CH, CH)],
                                ridx)
                pltpu.sync_copy(col_hbm.at[pl.ds(ebase + h * CH, CH)],
                                cidx)
                for b in range(3):
                    pltpu.async_copy(xs.at[ridx.at[b]], gb[b], gsem[b])

                def step(g, _):
                    for u in range(4):
                        b = g * 4 + u
                        pltpu.make_async_copy(xs.at[ridx.at[b]], gb[u],
                                              gsem[u]).wait()
                        pltpu.async_copy(gb[u], acc.at[cidx.at[b]],
                                         ssem[u], add=True)
                        un = (u + 3) % 4

                        @pl.when(b + 3 < CH)
                        def _():
                            @pl.when(b >= 1)
                            def _():
                                pltpu.make_async_copy(
                                    gb[un], acc.at[cidx.at[b]],
                                    ssem[un]).wait()

                            pltpu.async_copy(xs.at[ridx.at[b + 3]],
                                             gb[un], gsem[un])
                    return 0

                lax.fori_loop(0, CH // 4, step, 0)
                for u in range(4):
                    pltpu.make_async_copy(gb[u], acc.at[cidx.at[0]],
                                          ssem[u]).wait()

            plsc.subcore_barrier()
            obase = (2 * j + c) * NPAD + s * RPT
            pltpu.sync_copy(acc.at[pl.ds(s * RPT, RPT)],
                            t_hbm.at[pl.ds(obase, RPT)])

    return agg_kernel(xs0, xs1, xs2, row50, col50)


def _gate_call(tt, xs0, xs1, xs2, degp, hp,
               bz2, br2, bh2, wzl, wrl, whl, bzl2, brl2, bhl2):
    def body(t0a, t0b, t1a, t1b, t2a, t2b, x0, x1, x2, da, db, h,
             bz_r, br_r, bh_r, wz_r, wr_r, wh_r, bzl_r, brl_r, bhl_r, out):
        deg = da[...] + db[...] + 1.0
        dis = lax.rsqrt(deg)
        h_ = h[...]
        f32 = lambda v: v[...].astype(jnp.float32)
        cz = dis * (f32(t0a) + f32(t0b) + f32(x0)) + bz_r[...]
        cr = dis * (f32(t1a) + f32(t1b) + f32(x1)) + br_r[...]
        ch = dis * (f32(t2a) + f32(t2b) + f32(x2)) + bh_r[...]
        wz = wz_r[...]
        wr = wr_r[...]
        wh = wh_r[...]
        dot = lambda a, b: jnp.dot(a, b, preferred_element_type=jnp.float32)
        z = jax.nn.sigmoid(dot(cz, wz[:128]) + dot(h_, wz[128:]) + bzl_r[...])
        r = jax.nn.sigmoid(dot(cr, wr[:128]) + dot(h_, wr[128:]) + brl_r[...])
        ht = jnp.tanh(dot(ch, wh[:128]) + dot(h_ * r, wh[128:]) + bhl_r[...])
        out[...] = z * h_ + (1.0 - z) * ht

    grid = NPAD // RPT
    rb = lambda i: (i, 0)
    full = lambda i: (0, 0)
    tspec = lambda k: pl.BlockSpec((RPT, 128), lambda i, k=k: (i + k * grid, 0))
    return pl.pallas_call(
        body,
        grid=(grid,),
        in_specs=[
            tspec(0), tspec(1), tspec(2), tspec(3), tspec(4), tspec(5),
            pl.BlockSpec((RPT, 128), rb),
            pl.BlockSpec((RPT, 128), rb),
            pl.BlockSpec((RPT, 128), rb),
            pl.BlockSpec((RPT, 1), rb),
            pl.BlockSpec((RPT, 1), lambda i: (i + grid, 0)),
            pl.BlockSpec((RPT, 128), rb),
            pl.BlockSpec((1, 128), full),
            pl.BlockSpec((1, 128), full),
            pl.BlockSpec((1, 128), full),
            pl.BlockSpec((256, 128), full),
            pl.BlockSpec((256, 128), full),
            pl.BlockSpec((256, 128), full),
            pl.BlockSpec((1, 128), full),
            pl.BlockSpec((1, 128), full),
            pl.BlockSpec((1, 128), full),
        ],
        out_specs=pl.BlockSpec((RPT, 128), rb),
        out_shape=jax.ShapeDtypeStruct((NN, 128), jnp.float32),
    )(tt, tt, tt, tt, tt, tt, xs0, xs1, xs2, degp, degp, hp,
      bz2, br2, bh2, wzl, wrl, whl, bzl2, brl2, bhl2)


def kernel(x_t, h_prev, edge_index, Wz, bz, Wr, br, Wh, bh,
           Wz_lin, bz_lin, Wr_lin, br_lin, Wh_lin, bh_lin):
    row = edge_index[0]
    col = edge_index[1]
    col125 = col.reshape(EE // EB, EB)
    row50 = row.reshape(EE // EBA, EBA)
    col50 = col.reshape(EE // EBA, EBA)
    wcat = jnp.concatenate([Wz, Wr, Wh], axis=1)

    degp = _deg_call(col125).reshape(2 * NPAD, 1)
    xs0, xs1, xs2 = _scale_matmul_call(x_t, wcat, degp)
    tt = _agg_call(xs0, xs1, xs2, row50, col50)
    return _gate_call(
        tt, xs0, xs1, xs2, degp, h_prev,
        bz.reshape(1, 128), br.reshape(1, 128), bh.reshape(1, 128),
        Wz_lin, Wr_lin, Wh_lin,
        bz_lin.reshape(1, 128), br_lin.reshape(1, 128),
        bh_lin.reshape(1, 128))


# speedup vs baseline: 28.7549x; 1.0756x over previous
import functools

import jax
import jax.numpy as jnp
from jax import lax
from jax.experimental import pallas as pl
from jax.experimental.pallas import tpu as pltpu
from jax.experimental.pallas import tpu_sc as plsc

NN = 10000
EE = 320000
NPAD = 10240
RPT = 640
EB = 125
ERDEG = 80
ERAGG = 80
EBA = 50


def _deg_call(col125):
    mesh = plsc.VectorSubcoreMesh(core_axis_name="c", subcore_axis_name="s")

    @functools.partial(
        pl.kernel,
        out_type=jax.ShapeDtypeStruct((2 * NPAD,), jnp.float32),
        mesh=mesh,
        scratch_types=[
            pltpu.VMEM((ERDEG, EB), jnp.int32),
            pltpu.VMEM((RPT,), jnp.float32),
            pltpu.VMEM((128,), jnp.float32),
            pltpu.SemaphoreType.DMA,
            pltpu.VMEM_SHARED((NPAD,), jnp.float32),
        ],
    )
    def deg_kernel(col_hbm, degp_hbm, idx_v, zbuf_v, ones_v, sem, acc):
        c = lax.axis_index("c")
        s = lax.axis_index("s")
        w = s * 2 + c

        def zrow(i, _):
            zbuf_v[pl.ds(i * 16, 16)] = jnp.zeros((16,), jnp.float32)
            return 0

        lax.fori_loop(0, RPT // 16, zrow, 0)

        def orow(i, _):
            ones_v[pl.ds(i * 16, 16)] = jnp.ones((16,), jnp.float32)
            return 0

        lax.fori_loop(0, 8, orow, 0)

        pltpu.sync_copy(zbuf_v, acc.at[pl.ds(s * RPT, RPT)])
        plsc.subcore_barrier()

        pltpu.sync_copy(col_hbm.at[pl.ds(w * ERDEG, ERDEG)], idx_v)
        copies = [
            pltpu.async_copy(ones_v.at[pl.ds(0, EB)], acc.at[idx_v.at[b]],
                             sem, add=True)
            for b in range(ERDEG)
        ]
        for cp in copies:
            cp.wait()
        plsc.subcore_barrier()
        pltpu.sync_copy(acc.at[pl.ds(s * RPT, RPT)],
                        degp_hbm.at[pl.ds(c * NPAD + s * RPT, RPT)])

    return deg_kernel(col125)


def _scale_matmul_call(xp, wcat, degp):
    def body(x_ref, w_ref, da_ref, db_ref, o0_ref, o1_ref, o2_ref):
        deg = da_ref[...] + db_ref[...] + 1.0
        dis = lax.rsqrt(deg)
        res = jnp.dot(x_ref[...], w_ref[...],
                      preferred_element_type=jnp.float32)
        res = res * dis
        o0_ref[...] = res[:, 0:128]
        o1_ref[...] = res[:, 128:256]
        o2_ref[...] = res[:, 256:384]

    grid = NPAD // RPT
    rb = lambda i: (i, 0)
    return pl.pallas_call(
        body,
        grid=(grid,),
        in_specs=[
            pl.BlockSpec((RPT, 128), rb),
            pl.BlockSpec((128, 384), lambda i: (0, 0)),
            pl.BlockSpec((RPT, 1), rb),
            pl.BlockSpec((RPT, 1), lambda i: (i + grid, 0)),
        ],
        out_specs=[
            pl.BlockSpec((RPT, 128), rb),
            pl.BlockSpec((RPT, 128), rb),
            pl.BlockSpec((RPT, 128), rb),
        ],
        out_shape=[
            jax.ShapeDtypeStruct((NPAD, 128), jnp.float32),
            jax.ShapeDtypeStruct((NPAD, 128), jnp.float32),
            jax.ShapeDtypeStruct((NPAD, 128), jnp.float32),
        ],
    )(xp, wcat, degp, degp)


def _agg_call(xs0, xs1, xs2, row50, col50):
    mesh = plsc.VectorSubcoreMesh(core_axis_name="c", subcore_axis_name="s")
    NBH = EE // EBA // 2 // 16
    NBF = 2 * NBH
    CH = 40
    NBUF = 5

    @functools.partial(
        pl.kernel,
        out_type=jax.ShapeDtypeStruct((4 * NPAD, 128), jnp.float32),
        mesh=mesh,
        scratch_types=[
            pltpu.VMEM((CH, EBA), jnp.int32),
            pltpu.VMEM((CH, EBA), jnp.int32),
            [pltpu.VMEM((EBA, 128), jnp.float32) for _ in range(NBUF)],
            [pltpu.SemaphoreType.DMA for _ in range(NBUF)],
            [pltpu.SemaphoreType.DMA for _ in range(NBUF)],
            pltpu.VMEM_SHARED((NPAD, 128), jnp.float32),
        ],
    )
    def agg_kernel(x0_hbm, x1_hbm, x2_hbm, row_hbm, col_hbm, t_hbm,
                   ridx, cidx, gb, gsem, ssem, acc):
        c = lax.axis_index("c")
        s = lax.axis_index("s")

        def run(xs, nb, ebase, slot):
            def zrow(i, _):
                def zcol(k, _):
                    gb[0][i, pl.ds(k * 16, 16)] = jnp.zeros((16,),
                                                            jnp.float32)
                    return 0
                lax.fori_loop(0, 8, zcol, 0)
                return 0

            lax.fori_loop(0, EBA, zrow, 0)
            for k in range(RPT // 40):
                pltpu.sync_copy(gb[0].at[pl.ds(0, 40)],
                                acc.at[pl.ds(s * RPT + k * 40, 40)])
            plsc.subcore_barrier()

            for h in range(nb // CH):
                pltpu.sync_copy(row_hbm.at[pl.ds(ebase + h * CH, CH)],
                                ridx)
                pltpu.sync_copy(col_hbm.at[pl.ds(ebase + h * CH, CH)],
                                cidx)
                for b in range(NBUF - 1):
                    pltpu.async_copy(xs.at[ridx.at[b]], gb[b], gsem[b])

                def step(g, _):
                    for u in range(NBUF):
                        b = g * NBUF + u
                        pltpu.make_async_copy(xs.at[ridx.at[b]], gb[u],
                                              gsem[u]).wait()
                        pltpu.async_copy(gb[u], acc.at[cidx.at[b]],
                                         ssem[u], add=True)
                        un = (u + NBUF - 1) % NBUF

                        @pl.when(b + NBUF - 1 < CH)
                        def _():
                            @pl.when(b >= 1)
                            def _():
                                pltpu.make_async_copy(
                                    gb[un], acc.at[cidx.at[b]],
                                    ssem[un]).wait()

                            pltpu.async_copy(xs.at[ridx.at[b + NBUF - 1]],
                                             gb[un], gsem[un])
                    return 0

                lax.fori_loop(0, CH // NBUF, step, 0)
                for u in range(NBUF):
                    pltpu.make_async_copy(gb[u], acc.at[cidx.at[0]],
                                          ssem[u]).wait()

            plsc.subcore_barrier()
            obase = slot * NPAD + s * RPT
            pltpu.sync_copy(acc.at[pl.ds(s * RPT, RPT)],
                            t_hbm.at[pl.ds(obase, RPT)])

        @pl.when(c == 0)
        def _():
            run(x0_hbm, NBF, s * NBF, 0)
            run(x2_hbm, NBH, s * NBH, 2)

        @pl.when(c == 1)
        def _():
            run(x1_hbm, NBF, s * NBF, 1)
            run(x2_hbm, NBH, 16 * NBH + s * NBH, 3)

    return agg_kernel(xs0, xs1, xs2, row50, col50)


def _gate_call(tt, xs0, xs1, xs2, degp, hp,
               bz2, br2, bh2, wzl, wrl, whl, bzl2, brl2, bhl2):
    def body(t0, t1, t2a, t2b, x0, x1, x2, da, db, h,
             bz_r, br_r, bh_r, wz_r, wr_r, wh_r, bzl_r, brl_r, bhl_r, out):
        deg = da[...] + db[...] + 1.0
        dis = lax.rsqrt(deg)
        h_ = h[...]
        cz = dis * (t0[...] + x0[...]) + bz_r[...]
        cr = dis * (t1[...] + x1[...]) + br_r[...]
        ch = dis * (t2a[...] + t2b[...] + x2[...]) + bh_r[...]
        wz = wz_r[...]
        wr = wr_r[...]
        wh = wh_r[...]
        dot = lambda a, b: jnp.dot(a, b, preferred_element_type=jnp.float32)
        z = jax.nn.sigmoid(dot(cz, wz[:128]) + dot(h_, wz[128:]) + bzl_r[...])
        r = jax.nn.sigmoid(dot(cr, wr[:128]) + dot(h_, wr[128:]) + brl_r[...])
        ht = jnp.tanh(dot(ch, wh[:128]) + dot(h_ * r, wh[128:]) + bhl_r[...])
        out[...] = z * h_ + (1.0 - z) * ht

    grid = NPAD // RPT
    rb = lambda i: (i, 0)
    full = lambda i: (0, 0)
    tspec = lambda k: pl.BlockSpec((RPT, 128), lambda i, k=k: (i + k * grid, 0))
    return pl.pallas_call(
        body,
        grid=(grid,),
        in_specs=[
            tspec(0), tspec(1), tspec(2), tspec(3),
            pl.BlockSpec((RPT, 128), rb),
            pl.BlockSpec((RPT, 128), rb),
            pl.BlockSpec((RPT, 128), rb),
            pl.BlockSpec((RPT, 1), rb),
            pl.BlockSpec((RPT, 1), lambda i: (i + grid, 0)),
            pl.BlockSpec((RPT, 128), rb),
            pl.BlockSpec((1, 128), full),
            pl.BlockSpec((1, 128), full),
            pl.BlockSpec((1, 128), full),
            pl.BlockSpec((256, 128), full),
            pl.BlockSpec((256, 128), full),
            pl.BlockSpec((256, 128), full),
            pl.BlockSpec((1, 128), full),
            pl.BlockSpec((1, 128), full),
            pl.BlockSpec((1, 128), full),
        ],
        out_specs=pl.BlockSpec((RPT, 128), rb),
        out_shape=jax.ShapeDtypeStruct((NN, 128), jnp.float32),
    )(tt, tt, tt, tt, xs0, xs1, xs2, degp, degp, hp,
      bz2, br2, bh2, wzl, wrl, whl, bzl2, brl2, bhl2)


def kernel(x_t, h_prev, edge_index, Wz, bz, Wr, br, Wh, bh,
           Wz_lin, bz_lin, Wr_lin, br_lin, Wh_lin, bh_lin):
    row = edge_index[0]
    col = edge_index[1]
    col125 = col.reshape(EE // EB, EB)
    row50 = row.reshape(EE // EBA, EBA)
    col50 = col.reshape(EE // EBA, EBA)
    wcat = jnp.concatenate([Wz, Wr, Wh], axis=1)

    degp = _deg_call(col125).reshape(2 * NPAD, 1)
    xs0, xs1, xs2 = _scale_matmul_call(x_t, wcat, degp)
    tt = _agg_call(xs0, xs1, xs2, row50, col50)
    return _gate_call(
        tt, xs0, xs1, xs2, degp, h_prev,
        bz.reshape(1, 128), br.reshape(1, 128), bh.reshape(1, 128),
        Wz_lin, Wr_lin, Wh_lin,
        bz_lin.reshape(1, 128), br_lin.reshape(1, 128),
        bh_lin.reshape(1, 128))
